# baseline (speedup 1.0000x reference)
import functools

import jax
import jax.numpy as jnp
from jax import lax
from jax.experimental import pallas as pl
from jax.experimental.pallas import tpu as pltpu
from jax.experimental.pallas import tpu_sc as plsc

D = 128
K_NBS = 32
NS = 16
CHUNK = 8
VPR = D // 16
ROWS = CHUNK * K_NBS
BP = 10240
Q0 = 640
Q1 = BP // NS - Q0


def _accumulate_chunk(rows_v, pooled_v):
    for t in range(CHUNK):
        r0 = t * K_NBS

        def nb_body(i, a):
            for u in range(4):
                r = r0 + i * 4 + u
                a = tuple(a[v] + rows_v[r, pl.ds(v * 16, 16)]
                          for v in range(VPR))
            return a

        zero = jnp.zeros((16,), jnp.float32)
        accs = lax.fori_loop(0, K_NBS // 4, nb_body, (zero,) * VPR)
        for v in range(VPR):
            pooled_v[t, pl.ds(v * 16, 16)] = accs[v]


def _sc_gather_pool(x, nodes_p, nbs_flat):
    qmax = max(Q0, Q1)
    mesh = plsc.VectorSubcoreMesh(core_axis_name="c", subcore_axis_name="s")

    @functools.partial(
        pl.kernel,
        mesh=mesh,
        out_type=[
            jax.ShapeDtypeStruct((BP, D), jnp.float32),
            jax.ShapeDtypeStruct((BP, D), jnp.float32),
        ],
        scratch_types=[
            pltpu.VMEM((qmax,), jnp.int32),
            pltpu.VMEM((qmax * K_NBS,), jnp.int32),
            pltpu.VMEM((ROWS, D), jnp.float32),
            pltpu.VMEM((ROWS, D), jnp.float32),
            pltpu.VMEM((CHUNK, D), jnp.float32),
            pltpu.VMEM((CHUNK, D), jnp.float32),
            pltpu.VMEM((CHUNK, D), jnp.float32),
            pltpu.VMEM((CHUNK, D), jnp.float32),
            pltpu.SemaphoreType.DMA,
            pltpu.SemaphoreType.DMA,
            pltpu.SemaphoreType.DMA,
            pltpu.SemaphoreType.DMA,
            pltpu.SemaphoreType.DMA,
            pltpu.SemaphoreType.DMA,
            pltpu.SemaphoreType.DMA,
            pltpu.SemaphoreType.DMA,
        ],
    )
    def sc_kernel(x_hbm, nodes_hbm, nbs_hbm, self_hbm, pooled_hbm,
                  nid_v, nbid_v, rows0, rows1, pooled0, pooled1, selfb0, selfb1,
                  sem_g0, sem_g1, sem_p0, sem_p1,
                  sem_sg0, sem_sg1, sem_sw0, sem_sw1):
        cid = lax.axis_index("c")
        sid = lax.axis_index("s")
        rows = (rows0, rows1)
        pooled = (pooled0, pooled1)
        selfb = (selfb0, selfb1)
        sem_g = (sem_g0, sem_g1)
        sem_p = (sem_p0, sem_p1)
        sem_sg = (sem_sg0, sem_sg1)
        sem_sw = (sem_sw0, sem_sw1)

        def worker(q, wbase):
            n_chunks = q // CHUNK
            n_pairs = n_chunks // 2
            pltpu.sync_copy(nodes_hbm.at[pl.ds(wbase, q)],
                            nid_v.at[pl.ds(0, q)])
            pltpu.sync_copy(nbs_hbm.at[pl.ds(wbase * K_NBS, q * K_NBS)],
                            nbid_v.at[pl.ds(0, q * K_NBS)])

            def start_gathers(c, par):
                pltpu.async_copy(
                    x_hbm.at[nbid_v.at[pl.ds(c * ROWS, ROWS)]],
                    rows[par], sem_g[par])
                pltpu.async_copy(
                    x_hbm.at[nid_v.at[pl.ds(c * CHUNK, CHUNK)]],
                    selfb[par], sem_sg[par])

            def wait_gather(par):
                pltpu.make_async_copy(
                    x_hbm.at[nbid_v.at[pl.ds(0, ROWS)]],
                    rows[par], sem_g[par]).wait()

            def finish_chunk(c, par, first):
                wait_gather(par)

                @pl.when(jnp.logical_not(first))
                def _():
                    pltpu.make_async_copy(
                        pooled[par], pooled_hbm.at[pl.ds(wbase, CHUNK)],
                        sem_p[par]).wait()

                _accumulate_chunk(rows[par], pooled[par])
                pltpu.async_copy(
                    pooled[par],
                    pooled_hbm.at[pl.ds(wbase + c * CHUNK, CHUNK)],
                    sem_p[par])
                pltpu.make_async_copy(
                    x_hbm.at[nid_v.at[pl.ds(0, CHUNK)]],
                    selfb[par], sem_sg[par]).wait()

                @pl.when(jnp.logical_not(first))
                def _():
                    pltpu.make_async_copy(
                        selfb[par], self_hbm.at[pl.ds(wbase, CHUNK)],
                        sem_sw[par]).wait()

                pltpu.async_copy(
                    selfb[par],
                    self_hbm.at[pl.ds(wbase + c * CHUNK, CHUNK)],
                    sem_sw[par])

            start_gathers(0, 0)

            def pair_body(p, carry):
                c0 = p * 2

                @pl.when(c0 + 1 < n_chunks)
                def _():
                    start_gathers(c0 + 1, 1)

                finish_chunk(c0, 0, p == 0)

                @pl.when(c0 + 2 < n_chunks)
                def _():
                    start_gathers(c0 + 2, 0)

                finish_chunk(c0 + 1, 1, p == 0)
                return carry

            lax.fori_loop(0, n_pairs, pair_body, 0)
            for par in range(2):
                pltpu.make_async_copy(
                    pooled[par], pooled_hbm.at[pl.ds(wbase, CHUNK)],
                    sem_p[par]).wait()
                pltpu.make_async_copy(
                    selfb[par], self_hbm.at[pl.ds(wbase, CHUNK)],
                    sem_sw[par]).wait()

        @pl.when(cid == 0)
        def _():
            worker(Q0, sid * Q0)

        if Q1 > 0:
            @pl.when(cid == 1)
            def _():
                worker(Q1, NS * Q0 + sid * Q1)

    return sc_kernel(x, nodes_p, nbs_flat)


def _tc_combine_body(self_ref, pooled_ref, ws_ref, wn_ref, b_ref, o_ref):
    acc = jnp.dot(self_ref[...], ws_ref[...], preferred_element_type=jnp.float32)
    acc = acc + jnp.dot(pooled_ref[...], wn_ref[...],
                        preferred_element_type=jnp.float32)
    o_ref[...] = jnp.maximum(acc + b_ref[...], 0.0)


def _tc_combine(self_rows, pooled, ws, wn_scaled, bias2d):
    bk = 1280
    grid = (BP // bk,)
    return pl.pallas_call(
        _tc_combine_body,
        grid=grid,
        in_specs=[
            pl.BlockSpec((bk, D), lambda i: (i, 0)),
            pl.BlockSpec((bk, D), lambda i: (i, 0)),
            pl.BlockSpec((D, D), lambda i: (0, 0)),
            pl.BlockSpec((D, D), lambda i: (0, 0)),
            pl.BlockSpec((1, D), lambda i: (0, 0)),
        ],
        out_specs=pl.BlockSpec((bk, D), lambda i: (i, 0)),
        out_shape=jax.ShapeDtypeStruct((BP, D), jnp.float32),
    )(self_rows, pooled, ws, wn_scaled, bias2d)


def kernel(x, nodes, nbs_idx, self_weight, nb_weight, bias):
    b = nodes.shape[0]
    nodes_p = jnp.pad(nodes.astype(jnp.int32), (0, BP - b))
    nbs_flat = jnp.pad(nbs_idx.astype(jnp.int32), ((0, BP - b), (0, 0))).reshape(-1)
    self_rows, pooled = _sc_gather_pool(x, nodes_p, nbs_flat)
    out = _tc_combine(self_rows, pooled, self_weight,
                      nb_weight * (1.0 / K_NBS), bias.reshape(1, D))
    return out[:b]


# speedup vs baseline: 4.9452x; 4.9452x over previous
import functools

import jax
import jax.numpy as jnp
from jax import lax
from jax.experimental import pallas as pl
from jax.experimental.pallas import tpu as pltpu
from jax.experimental.pallas import tpu_sc as plsc

D = 128
K_NBS = 32
NS = 16
CHUNK = 8
VPR = D // 16
ROWS = CHUNK * K_NBS
BP = 10240
Q0 = 320
Q1 = BP // NS - Q0


def _accumulate_chunk(rows_v, pooled_v):
    for t in range(CHUNK):
        r0 = t * K_NBS

        def nb_body(i, a):
            for u in range(4):
                r = r0 + i * 4 + u
                a = tuple(a[v] + rows_v[r, pl.ds(v * 16, 16)]
                          for v in range(VPR))
            return a

        zero = jnp.zeros((16,), jnp.float32)
        accs = lax.fori_loop(0, K_NBS // 4, nb_body, (zero,) * VPR)
        for v in range(VPR):
            pooled_v[t, pl.ds(v * 16, 16)] = accs[v]


def _sc_gather_pool(x, nodes_p, nbs_flat):
    qmax = max(Q0, Q1)
    mesh = plsc.VectorSubcoreMesh(core_axis_name="c", subcore_axis_name="s")

    @functools.partial(
        pl.kernel,
        mesh=mesh,
        out_type=[
            jax.ShapeDtypeStruct((BP, D), jnp.float32),
            jax.ShapeDtypeStruct((BP, D), jnp.float32),
        ],
        scratch_types=[
            pltpu.VMEM((qmax,), jnp.int32),
            pltpu.VMEM((qmax * K_NBS,), jnp.int32),
            pltpu.VMEM((ROWS, D), jnp.float32),
            pltpu.VMEM((ROWS, D), jnp.float32),
            pltpu.VMEM((CHUNK, D), jnp.float32),
            pltpu.VMEM((CHUNK, D), jnp.float32),
            pltpu.VMEM((CHUNK, D), jnp.float32),
            pltpu.VMEM((CHUNK, D), jnp.float32),
            pltpu.SemaphoreType.DMA,
            pltpu.SemaphoreType.DMA,
            pltpu.SemaphoreType.DMA,
            pltpu.SemaphoreType.DMA,
            pltpu.SemaphoreType.DMA,
            pltpu.SemaphoreType.DMA,
            pltpu.SemaphoreType.DMA,
            pltpu.SemaphoreType.DMA,
        ],
    )
    def sc_kernel(x_hbm, nodes_hbm, nbs_hbm, self_hbm, pooled_hbm,
                  nid_v, nbid_v, rows0, rows1, pooled0, pooled1, selfb0, selfb1,
                  sem_g0, sem_g1, sem_p0, sem_p1,
                  sem_sg0, sem_sg1, sem_sw0, sem_sw1):
        cid = lax.axis_index("c")
        sid = lax.axis_index("s")
        rows = (rows0, rows1)
        pooled = (pooled0, pooled1)
        selfb = (selfb0, selfb1)
        sem_g = (sem_g0, sem_g1)
        sem_p = (sem_p0, sem_p1)
        sem_sg = (sem_sg0, sem_sg1)
        sem_sw = (sem_sw0, sem_sw1)

        def worker(q, wbase):
            n_chunks = q // CHUNK
            n_pairs = n_chunks // 2
            pltpu.sync_copy(nodes_hbm.at[pl.ds(wbase, q)],
                            nid_v.at[pl.ds(0, q)])
            pltpu.sync_copy(nbs_hbm.at[pl.ds(wbase * K_NBS, q * K_NBS)],
                            nbid_v.at[pl.ds(0, q * K_NBS)])

            def start_gathers(c, par):
                pltpu.async_copy(
                    x_hbm.at[nbid_v.at[pl.ds(c * ROWS, ROWS)]],
                    rows[par], sem_g[par])
                pltpu.async_copy(
                    x_hbm.at[nid_v.at[pl.ds(c * CHUNK, CHUNK)]],
                    selfb[par], sem_sg[par])

            def wait_gather(par):
                pltpu.make_async_copy(
                    x_hbm.at[nbid_v.at[pl.ds(0, ROWS)]],
                    rows[par], sem_g[par]).wait()

            def finish_chunk(c, par, first):
                wait_gather(par)

                @pl.when(jnp.logical_not(first))
                def _():
                    pltpu.make_async_copy(
                        pooled[par], pooled_hbm.at[pl.ds(wbase, CHUNK)],
                        sem_p[par]).wait()

                _accumulate_chunk(rows[par], pooled[par])
                pltpu.async_copy(
                    pooled[par],
                    pooled_hbm.at[pl.ds(wbase + c * CHUNK, CHUNK)],
                    sem_p[par])
                pltpu.make_async_copy(
                    x_hbm.at[nid_v.at[pl.ds(0, CHUNK)]],
                    selfb[par], sem_sg[par]).wait()

                @pl.when(jnp.logical_not(first))
                def _():
                    pltpu.make_async_copy(
                        selfb[par], self_hbm.at[pl.ds(wbase, CHUNK)],
                        sem_sw[par]).wait()

                pltpu.async_copy(
                    selfb[par],
                    self_hbm.at[pl.ds(wbase + c * CHUNK, CHUNK)],
                    sem_sw[par])

            start_gathers(0, 0)

            def pair_body(p, carry):
                c0 = p * 2

                @pl.when(c0 + 1 < n_chunks)
                def _():
                    start_gathers(c0 + 1, 1)

                finish_chunk(c0, 0, p == 0)

                @pl.when(c0 + 2 < n_chunks)
                def _():
                    start_gathers(c0 + 2, 0)

                finish_chunk(c0 + 1, 1, p == 0)
                return carry

            lax.fori_loop(0, n_pairs, pair_body, 0)
            for par in range(2):
                pltpu.make_async_copy(
                    pooled[par], pooled_hbm.at[pl.ds(wbase, CHUNK)],
                    sem_p[par]).wait()
                pltpu.make_async_copy(
                    selfb[par], self_hbm.at[pl.ds(wbase, CHUNK)],
                    sem_sw[par]).wait()

        @pl.when(cid == 0)
        def _():
            worker(Q0, sid * Q0)

        if Q1 > 0:
            @pl.when(cid == 1)
            def _():
                worker(Q1, NS * Q0 + sid * Q1)

    return sc_kernel(x, nodes_p, nbs_flat)


def _tc_combine_body(self_ref, pooled_ref, ws_ref, wn_ref, b_ref, o_ref):
    acc = jnp.dot(self_ref[...], ws_ref[...], preferred_element_type=jnp.float32)
    acc = acc + jnp.dot(pooled_ref[...], wn_ref[...],
                        preferred_element_type=jnp.float32)
    o_ref[...] = jnp.maximum(acc + b_ref[...], 0.0)


def _tc_combine(self_rows, pooled, ws, wn_scaled, bias2d):
    bk = 1280
    grid = (BP // bk,)
    return pl.pallas_call(
        _tc_combine_body,
        grid=grid,
        in_specs=[
            pl.BlockSpec((bk, D), lambda i: (i, 0)),
            pl.BlockSpec((bk, D), lambda i: (i, 0)),
            pl.BlockSpec((D, D), lambda i: (0, 0)),
            pl.BlockSpec((D, D), lambda i: (0, 0)),
            pl.BlockSpec((1, D), lambda i: (0, 0)),
        ],
        out_specs=pl.BlockSpec((bk, D), lambda i: (i, 0)),
        out_shape=jax.ShapeDtypeStruct((BP, D), jnp.float32),
    )(self_rows, pooled, ws, wn_scaled, bias2d)


def kernel(x, nodes, nbs_idx, self_weight, nb_weight, bias):
    b = nodes.shape[0]
    n = x.shape[0]
    pad_n = jnp.arange(BP - b, dtype=jnp.int32) % n
    pad_nb = jnp.arange((BP - b) * K_NBS, dtype=jnp.int32) % n
    nodes_p = jnp.concatenate([nodes.astype(jnp.int32), pad_n])
    nbs_flat = jnp.concatenate(
        [nbs_idx.astype(jnp.int32).reshape(-1), pad_nb])
    self_rows, pooled = _sc_gather_pool(x, nodes_p, nbs_flat)
    out = _tc_combine(self_rows, pooled, self_weight,
                      nb_weight * (1.0 / K_NBS), bias.reshape(1, D))
    return out[:b]


# speedup vs baseline: 5.2263x; 1.0569x over previous
import functools

import jax
import jax.numpy as jnp
from jax import lax
from jax.experimental import pallas as pl
from jax.experimental.pallas import tpu as pltpu
from jax.experimental.pallas import tpu_sc as plsc

D = 128
K_NBS = 32
NS = 16
NW = 2 * NS
CHUNK = 8
VPR = D // 16
ROWS = CHUNK * K_NBS
B_REAL = 10000
BP = 10240
Q = BP // NW
Q_TAIL = B_REAL - (NW - 1) * Q


def _accumulate_chunk(rows_v, pooled_v):
    for t in range(CHUNK):
        r0 = t * K_NBS

        def nb_body(i, a):
            for u in range(4):
                r = r0 + i * 4 + u
                a = tuple(a[v] + rows_v[r, pl.ds(v * 16, 16)]
                          for v in range(VPR))
            return a

        zero = jnp.zeros((16,), jnp.float32)
        accs = lax.fori_loop(0, K_NBS // 4, nb_body, (zero,) * VPR)
        for v in range(VPR):
            pooled_v[t, pl.ds(v * 16, 16)] = accs[v]


def _sc_gather_pool(x, nodes, nbs_flat, pad_n, pad_nb):
    mesh = plsc.VectorSubcoreMesh(core_axis_name="c", subcore_axis_name="s")

    @functools.partial(
        pl.kernel,
        mesh=mesh,
        out_type=[
            jax.ShapeDtypeStruct((BP, D), jnp.float32),
            jax.ShapeDtypeStruct((BP, D), jnp.float32),
        ],
        scratch_types=[
            pltpu.VMEM((Q,), jnp.int32),
            pltpu.VMEM((Q * K_NBS,), jnp.int32),
            pltpu.VMEM((ROWS, D), jnp.float32),
            pltpu.VMEM((ROWS, D), jnp.float32),
            pltpu.VMEM((CHUNK, D), jnp.float32),
            pltpu.VMEM((CHUNK, D), jnp.float32),
            pltpu.VMEM((CHUNK, D), jnp.float32),
            pltpu.VMEM((CHUNK, D), jnp.float32),
            pltpu.SemaphoreType.DMA,
            pltpu.SemaphoreType.DMA,
            pltpu.SemaphoreType.DMA,
            pltpu.SemaphoreType.DMA,
            pltpu.SemaphoreType.DMA,
            pltpu.SemaphoreType.DMA,
            pltpu.SemaphoreType.DMA,
            pltpu.SemaphoreType.DMA,
        ],
    )
    def sc_kernel(x_hbm, nodes_hbm, nbs_hbm, padn_hbm, padnb_hbm,
                  self_hbm, pooled_hbm,
                  nid_v, nbid_v, rows0, rows1, pooled0, pooled1, selfb0, selfb1,
                  sem_g0, sem_g1, sem_p0, sem_p1,
                  sem_sg0, sem_sg1, sem_sw0, sem_sw1):
        cid = lax.axis_index("c")
        sid = lax.axis_index("s")
        wid = sid * 2 + cid
        wbase = wid * Q
        rows = (rows0, rows1)
        pooled = (pooled0, pooled1)
        selfb = (selfb0, selfb1)
        sem_g = (sem_g0, sem_g1)
        sem_p = (sem_p0, sem_p1)
        sem_sg = (sem_sg0, sem_sg1)
        sem_sw = (sem_sw0, sem_sw1)

        is_tail = wid == NW - 1

        @pl.when(jnp.logical_not(is_tail))
        def _():
            pltpu.sync_copy(nodes_hbm.at[pl.ds(wbase, Q)], nid_v)
            pltpu.sync_copy(nbs_hbm.at[pl.ds(wbase * K_NBS, Q * K_NBS)],
                            nbid_v)

        @pl.when(is_tail)
        def _():
            tb = (NW - 1) * Q
            pltpu.sync_copy(nodes_hbm.at[pl.ds(tb, Q_TAIL)],
                            nid_v.at[pl.ds(0, Q_TAIL)])
            pltpu.sync_copy(padn_hbm, nid_v.at[pl.ds(Q_TAIL, Q - Q_TAIL)])
            pltpu.sync_copy(nbs_hbm.at[pl.ds(tb * K_NBS, Q_TAIL * K_NBS)],
                            nbid_v.at[pl.ds(0, Q_TAIL * K_NBS)])
            pltpu.sync_copy(padnb_hbm,
                            nbid_v.at[pl.ds(Q_TAIL * K_NBS,
                                            (Q - Q_TAIL) * K_NBS)])

        def start_gathers(c, par):
            pltpu.async_copy(
                x_hbm.at[nbid_v.at[pl.ds(c * ROWS, ROWS)]],
                rows[par], sem_g[par])
            pltpu.async_copy(
                x_hbm.at[nid_v.at[pl.ds(c * CHUNK, CHUNK)]],
                selfb[par], sem_sg[par])

        def finish_chunk(c, par, first):
            pltpu.make_async_copy(
                x_hbm.at[nbid_v.at[pl.ds(0, ROWS)]],
                rows[par], sem_g[par]).wait()

            @pl.when(jnp.logical_not(first))
            def _():
                pltpu.make_async_copy(
                    pooled[par], pooled_hbm.at[pl.ds(wbase, CHUNK)],
                    sem_p[par]).wait()

            _accumulate_chunk(rows[par], pooled[par])
            pltpu.async_copy(
                pooled[par],
                pooled_hbm.at[pl.ds(wbase + c * CHUNK, CHUNK)],
                sem_p[par])
            pltpu.make_async_copy(
                x_hbm.at[nid_v.at[pl.ds(0, CHUNK)]],
                selfb[par], sem_sg[par]).wait()

            @pl.when(jnp.logical_not(first))
            def _():
                pltpu.make_async_copy(
                    selfb[par], self_hbm.at[pl.ds(wbase, CHUNK)],
                    sem_sw[par]).wait()

            pltpu.async_copy(
                selfb[par],
                self_hbm.at[pl.ds(wbase + c * CHUNK, CHUNK)],
                sem_sw[par])

        n_chunks = Q // CHUNK
        start_gathers(0, 0)

        def pair_body(p, carry):
            c0 = p * 2

            @pl.when(c0 + 1 < n_chunks)
            def _():
                start_gathers(c0 + 1, 1)

            finish_chunk(c0, 0, p == 0)

            @pl.when(c0 + 2 < n_chunks)
            def _():
                start_gathers(c0 + 2, 0)

            finish_chunk(c0 + 1, 1, p == 0)
            return carry

        lax.fori_loop(0, n_chunks // 2, pair_body, 0)
        for par in range(2):
            pltpu.make_async_copy(
                pooled[par], pooled_hbm.at[pl.ds(wbase, CHUNK)],
                sem_p[par]).wait()
            pltpu.make_async_copy(
                selfb[par], self_hbm.at[pl.ds(wbase, CHUNK)],
                sem_sw[par]).wait()

    return sc_kernel(x, nodes, nbs_flat, pad_n, pad_nb)


def _tc_combine_body(self_ref, pooled_ref, ws_ref, wn_ref, b_ref, o_ref):
    acc = jnp.dot(self_ref[...], ws_ref[...], preferred_element_type=jnp.float32)
    acc = acc + jnp.dot(pooled_ref[...], wn_ref[...],
                        preferred_element_type=jnp.float32)
    o_ref[...] = jnp.maximum(acc + b_ref[...], 0.0)


def _tc_combine(self_rows, pooled, ws, wn_scaled, bias2d):
    bk = 1000
    grid = (B_REAL // bk,)
    return pl.pallas_call(
        _tc_combine_body,
        grid=grid,
        in_specs=[
            pl.BlockSpec((bk, D), lambda i: (i, 0)),
            pl.BlockSpec((bk, D), lambda i: (i, 0)),
            pl.BlockSpec((D, D), lambda i: (0, 0)),
            pl.BlockSpec((D, D), lambda i: (0, 0)),
            pl.BlockSpec((1, D), lambda i: (0, 0)),
        ],
        out_specs=pl.BlockSpec((bk, D), lambda i: (i, 0)),
        out_shape=jax.ShapeDtypeStruct((B_REAL, D), jnp.float32),
    )(self_rows, pooled, ws, wn_scaled, bias2d)


def kernel(x, nodes, nbs_idx, self_weight, nb_weight, bias):
    n = x.shape[0]
    pad_n = jnp.arange(BP - B_REAL, dtype=jnp.int32) % n
    pad_nb = jnp.arange((BP - B_REAL) * K_NBS, dtype=jnp.int32) % n
    self_rows, pooled = _sc_gather_pool(
        x, nodes.astype(jnp.int32), nbs_idx.astype(jnp.int32).reshape(-1),
        pad_n, pad_nb)
    return _tc_combine(self_rows, pooled, self_weight,
                       nb_weight * (1.0 / K_NBS), bias.reshape(1, D))


# speedup vs baseline: 5.2469x; 1.0039x over previous
import functools

import jax
import jax.numpy as jnp
import numpy as np
from jax import lax
from jax.experimental import pallas as pl
from jax.experimental.pallas import tpu as pltpu
from jax.experimental.pallas import tpu_sc as plsc

D = 128
K_NBS = 32
NS = 16
NW = 2 * NS
CHUNK = 8
VPR = D // 16
ROWS = CHUNK * K_NBS
B_REAL = 10000
BP = 10240
Q = BP // NW
Q_TAIL = B_REAL - (NW - 1) * Q


def _accumulate_chunk(rows_v, pooled_v):
    for t in range(CHUNK):
        r0 = t * K_NBS

        def nb_body(i, a):
            for u in range(4):
                r = r0 + i * 4 + u
                a = tuple(a[v] + rows_v[r, pl.ds(v * 16, 16)]
                          for v in range(VPR))
            return a

        zero = jnp.zeros((16,), jnp.float32)
        accs = lax.fori_loop(0, K_NBS // 4, nb_body, (zero,) * VPR)
        for v in range(VPR):
            pooled_v[t, pl.ds(v * 16, 16)] = accs[v]


def _sc_gather_pool(x, nodes, nbs_flat, pad_n, pad_nb):
    mesh = plsc.VectorSubcoreMesh(core_axis_name="c", subcore_axis_name="s")

    @functools.partial(
        pl.kernel,
        mesh=mesh,
        out_type=[
            jax.ShapeDtypeStruct((BP, D), jnp.float32),
            jax.ShapeDtypeStruct((BP, D), jnp.float32),
        ],
        scratch_types=[
            pltpu.VMEM((Q,), jnp.int32),
            pltpu.VMEM((Q * K_NBS,), jnp.int32),
            pltpu.VMEM((ROWS, D), jnp.float32),
            pltpu.VMEM((ROWS, D), jnp.float32),
            pltpu.VMEM((CHUNK, D), jnp.float32),
            pltpu.VMEM((CHUNK, D), jnp.float32),
            pltpu.VMEM((CHUNK, D), jnp.float32),
            pltpu.VMEM((CHUNK, D), jnp.float32),
            pltpu.SemaphoreType.DMA,
            pltpu.SemaphoreType.DMA,
            pltpu.SemaphoreType.DMA,
            pltpu.SemaphoreType.DMA,
            pltpu.SemaphoreType.DMA,
            pltpu.SemaphoreType.DMA,
            pltpu.SemaphoreType.DMA,
            pltpu.SemaphoreType.DMA,
        ],
    )
    def sc_kernel(x_hbm, nodes_hbm, nbs_hbm, padn_hbm, padnb_hbm,
                  self_hbm, pooled_hbm,
                  nid_v, nbid_v, rows0, rows1, pooled0, pooled1, selfb0, selfb1,
                  sem_g0, sem_g1, sem_p0, sem_p1,
                  sem_sg0, sem_sg1, sem_sw0, sem_sw1):
        cid = lax.axis_index("c")
        sid = lax.axis_index("s")
        wid = sid * 2 + cid
        wbase = wid * Q
        rows = (rows0, rows1)
        pooled = (pooled0, pooled1)
        selfb = (selfb0, selfb1)
        sem_g = (sem_g0, sem_g1)
        sem_p = (sem_p0, sem_p1)
        sem_sg = (sem_sg0, sem_sg1)
        sem_sw = (sem_sw0, sem_sw1)

        is_tail = wid == NW - 1

        @pl.when(jnp.logical_not(is_tail))
        def _():
            pltpu.sync_copy(nodes_hbm.at[pl.ds(wbase, Q)], nid_v)
            pltpu.sync_copy(nbs_hbm.at[pl.ds(wbase * K_NBS, Q * K_NBS)],
                            nbid_v)

        @pl.when(is_tail)
        def _():
            tb = (NW - 1) * Q
            pltpu.sync_copy(nodes_hbm.at[pl.ds(tb, Q_TAIL)],
                            nid_v.at[pl.ds(0, Q_TAIL)])
            pltpu.sync_copy(padn_hbm, nid_v.at[pl.ds(Q_TAIL, Q - Q_TAIL)])
            pltpu.sync_copy(nbs_hbm.at[pl.ds(tb * K_NBS, Q_TAIL * K_NBS)],
                            nbid_v.at[pl.ds(0, Q_TAIL * K_NBS)])
            pltpu.sync_copy(padnb_hbm,
                            nbid_v.at[pl.ds(Q_TAIL * K_NBS,
                                            (Q - Q_TAIL) * K_NBS)])

        def start_gathers(c, par):
            pltpu.async_copy(
                x_hbm.at[nbid_v.at[pl.ds(c * ROWS, ROWS)]],
                rows[par], sem_g[par])
            pltpu.async_copy(
                x_hbm.at[nid_v.at[pl.ds(c * CHUNK, CHUNK)]],
                selfb[par], sem_sg[par])

        def finish_chunk(c, par, first):
            pltpu.make_async_copy(
                x_hbm.at[nbid_v.at[pl.ds(0, ROWS)]],
                rows[par], sem_g[par]).wait()

            @pl.when(jnp.logical_not(first))
            def _():
                pltpu.make_async_copy(
                    pooled[par], pooled_hbm.at[pl.ds(wbase, CHUNK)],
                    sem_p[par]).wait()

            _accumulate_chunk(rows[par], pooled[par])
            pltpu.async_copy(
                pooled[par],
                pooled_hbm.at[pl.ds(wbase + c * CHUNK, CHUNK)],
                sem_p[par])
            pltpu.make_async_copy(
                x_hbm.at[nid_v.at[pl.ds(0, CHUNK)]],
                selfb[par], sem_sg[par]).wait()

            @pl.when(jnp.logical_not(first))
            def _():
                pltpu.make_async_copy(
                    selfb[par], self_hbm.at[pl.ds(wbase, CHUNK)],
                    sem_sw[par]).wait()

            pltpu.async_copy(
                selfb[par],
                self_hbm.at[pl.ds(wbase + c * CHUNK, CHUNK)],
                sem_sw[par])

        n_chunks = Q // CHUNK
        start_gathers(0, 0)

        def pair_body(p, carry):
            c0 = p * 2

            @pl.when(c0 + 1 < n_chunks)
            def _():
                start_gathers(c0 + 1, 1)

            finish_chunk(c0, 0, p == 0)

            @pl.when(c0 + 2 < n_chunks)
            def _():
                start_gathers(c0 + 2, 0)

            finish_chunk(c0 + 1, 1, p == 0)
            return carry

        lax.fori_loop(0, n_chunks // 2, pair_body, 0)
        for par in range(2):
            pltpu.make_async_copy(
                pooled[par], pooled_hbm.at[pl.ds(wbase, CHUNK)],
                sem_p[par]).wait()
            pltpu.make_async_copy(
                selfb[par], self_hbm.at[pl.ds(wbase, CHUNK)],
                sem_sw[par]).wait()

    return sc_kernel(x, nodes, nbs_flat, pad_n, pad_nb)


def _tc_combine_body(self_ref, pooled_ref, ws_ref, wn_ref, b_ref, o_ref):
    acc = jnp.dot(self_ref[...], ws_ref[...], preferred_element_type=jnp.float32)
    acc = acc + jnp.dot(pooled_ref[...], wn_ref[...],
                        preferred_element_type=jnp.float32)
    o_ref[...] = jnp.maximum(acc + b_ref[...], 0.0)


def _tc_combine(self_rows, pooled, ws, wn_scaled, bias2d):
    bk = 1000
    grid = (B_REAL // bk,)
    return pl.pallas_call(
        _tc_combine_body,
        grid=grid,
        in_specs=[
            pl.BlockSpec((bk, D), lambda i: (i, 0)),
            pl.BlockSpec((bk, D), lambda i: (i, 0)),
            pl.BlockSpec((D, D), lambda i: (0, 0)),
            pl.BlockSpec((D, D), lambda i: (0, 0)),
            pl.BlockSpec((1, D), lambda i: (0, 0)),
        ],
        out_specs=pl.BlockSpec((bk, D), lambda i: (i, 0)),
        out_shape=jax.ShapeDtypeStruct((B_REAL, D), jnp.float32),
    )(self_rows, pooled, ws, wn_scaled, bias2d)


def kernel(x, nodes, nbs_idx, self_weight, nb_weight, bias):
    n = x.shape[0]
    pad_n = jnp.asarray(np.arange(BP - B_REAL, dtype=np.int32) % n)
    pad_nb = jnp.asarray(np.arange((BP - B_REAL) * K_NBS, dtype=np.int32) % n)
    self_rows, pooled = _sc_gather_pool(
        x, nodes.astype(jnp.int32), nbs_idx.astype(jnp.int32).reshape(-1),
        pad_n, pad_nb)
    return _tc_combine(self_rows, pooled, self_weight,
                       nb_weight * (1.0 / K_NBS), bias.reshape(1, D))


# speedup vs baseline: 5.3672x; 1.0229x over previous
import functools

import jax
import jax.numpy as jnp
import numpy as np
from jax import lax
from jax.experimental import pallas as pl
from jax.experimental.pallas import tpu as pltpu
from jax.experimental.pallas import tpu_sc as plsc

D = 128
K_NBS = 32
NS = 16
NW = 2 * NS
CHUNK = 8
VPR = D // 16
ROWS = CHUNK * K_NBS
B_REAL = 10000
BP = 10240
Q = BP // NW
Q_TAIL = B_REAL - (NW - 1) * Q


def _accumulate_chunk(rows_v, pooled_v):
    for t in range(CHUNK):
        r0 = t * K_NBS

        def nb_body(i, a):
            for u in range(4):
                r = r0 + i * 4 + u
                a = tuple(a[v] + rows_v[r, pl.ds(v * 16, 16)]
                          for v in range(VPR))
            return a

        zero = jnp.zeros((16,), jnp.float32)
        accs = lax.fori_loop(0, K_NBS // 4, nb_body, (zero,) * VPR)
        for v in range(VPR):
            pooled_v[t, pl.ds(v * 16, 16)] = accs[v]


def _sc_gather_pool(x, nodes, nbs_flat, pad_n, pad_nb):
    mesh = plsc.VectorSubcoreMesh(core_axis_name="c", subcore_axis_name="s")

    @functools.partial(
        pl.kernel,
        mesh=mesh,
        out_type=[
            jax.ShapeDtypeStruct((BP, D), jnp.float32),
            jax.ShapeDtypeStruct((BP, D), jnp.float32),
        ],
        scratch_types=[
            pltpu.VMEM((Q,), jnp.int32),
            pltpu.VMEM((Q, K_NBS), jnp.int32),
            pltpu.VMEM((Q * K_NBS,), jnp.int32),
            pltpu.VMEM((ROWS, D), jnp.float32),
            pltpu.VMEM((ROWS, D), jnp.float32),
            pltpu.VMEM((CHUNK, D), jnp.float32),
            pltpu.VMEM((CHUNK, D), jnp.float32),
            pltpu.VMEM((CHUNK, D), jnp.float32),
            pltpu.VMEM((CHUNK, D), jnp.float32),
            pltpu.SemaphoreType.DMA,
            pltpu.SemaphoreType.DMA,
            pltpu.SemaphoreType.DMA,
            pltpu.SemaphoreType.DMA,
            pltpu.SemaphoreType.DMA,
            pltpu.SemaphoreType.DMA,
            pltpu.SemaphoreType.DMA,
            pltpu.SemaphoreType.DMA,
        ],
    )
    def sc_kernel(x_hbm, nodes_hbm, nbs_hbm, padn_hbm, padnb_hbm,
                  self_hbm, pooled_hbm,
                  nid_v, nbid2_v, nbid_v, rows0, rows1,
                  pooled0, pooled1, selfb0, selfb1,
                  sem_g0, sem_g1, sem_p0, sem_p1,
                  sem_sg0, sem_sg1, sem_sw0, sem_sw1):
        cid = lax.axis_index("c")
        sid = lax.axis_index("s")
        wid = sid * 2 + cid
        wbase = wid * Q
        rows = (rows0, rows1)
        pooled = (pooled0, pooled1)
        selfb = (selfb0, selfb1)
        sem_g = (sem_g0, sem_g1)
        sem_p = (sem_p0, sem_p1)
        sem_sg = (sem_sg0, sem_sg1)
        sem_sw = (sem_sw0, sem_sw1)

        is_tail = wid == NW - 1

        @pl.when(jnp.logical_not(is_tail))
        def _():
            pltpu.sync_copy(nodes_hbm.at[pl.ds(wbase, Q)], nid_v)
            pltpu.sync_copy(nbs_hbm.at[pl.ds(wbase, Q)], nbid2_v)

        @pl.when(is_tail)
        def _():
            tb = (NW - 1) * Q
            pltpu.sync_copy(nodes_hbm.at[pl.ds(tb, Q_TAIL)],
                            nid_v.at[pl.ds(0, Q_TAIL)])
            pltpu.sync_copy(padn_hbm, nid_v.at[pl.ds(Q_TAIL, Q - Q_TAIL)])
            pltpu.sync_copy(nbs_hbm.at[pl.ds(tb, Q_TAIL)],
                            nbid2_v.at[pl.ds(0, Q_TAIL)])
            pltpu.sync_copy(padnb_hbm,
                            nbid2_v.at[pl.ds(Q_TAIL, Q - Q_TAIL)])

        def flatten_ids(lo, hi):
            def body(q, carry):
                for h in range(K_NBS // 16):
                    nbid_v[pl.ds(q * K_NBS + h * 16, 16)] = (
                        nbid2_v[q, pl.ds(h * 16, 16)])
                return carry

            lax.fori_loop(lo, hi, body, 0)

        def start_gathers(c, par):
            pltpu.async_copy(
                x_hbm.at[nbid_v.at[pl.ds(c * ROWS, ROWS)]],
                rows[par], sem_g[par])
            pltpu.async_copy(
                x_hbm.at[nid_v.at[pl.ds(c * CHUNK, CHUNK)]],
                selfb[par], sem_sg[par])

        def finish_chunk(c, par, first):
            pltpu.make_async_copy(
                x_hbm.at[nbid_v.at[pl.ds(0, ROWS)]],
                rows[par], sem_g[par]).wait()

            @pl.when(jnp.logical_not(first))
            def _():
                pltpu.make_async_copy(
                    pooled[par], pooled_hbm.at[pl.ds(wbase, CHUNK)],
                    sem_p[par]).wait()

            _accumulate_chunk(rows[par], pooled[par])
            pltpu.async_copy(
                pooled[par],
                pooled_hbm.at[pl.ds(wbase + c * CHUNK, CHUNK)],
                sem_p[par])
            pltpu.make_async_copy(
                x_hbm.at[nid_v.at[pl.ds(0, CHUNK)]],
                selfb[par], sem_sg[par]).wait()

            @pl.when(jnp.logical_not(first))
            def _():
                pltpu.make_async_copy(
                    selfb[par], self_hbm.at[pl.ds(wbase, CHUNK)],
                    sem_sw[par]).wait()

            pltpu.async_copy(
                selfb[par],
                self_hbm.at[pl.ds(wbase + c * CHUNK, CHUNK)],
                sem_sw[par])

        n_chunks = Q // CHUNK
        flatten_ids(0, CHUNK)
        start_gathers(0, 0)
        flatten_ids(CHUNK, Q)

        def pair_body(p, carry):
            c0 = p * 2

            @pl.when(c0 + 1 < n_chunks)
            def _():
                start_gathers(c0 + 1, 1)

            finish_chunk(c0, 0, p == 0)

            @pl.when(c0 + 2 < n_chunks)
            def _():
                start_gathers(c0 + 2, 0)

            finish_chunk(c0 + 1, 1, p == 0)
            return carry

        lax.fori_loop(0, n_chunks // 2, pair_body, 0)
        for par in range(2):
            pltpu.make_async_copy(
                pooled[par], pooled_hbm.at[pl.ds(wbase, CHUNK)],
                sem_p[par]).wait()
            pltpu.make_async_copy(
                selfb[par], self_hbm.at[pl.ds(wbase, CHUNK)],
                sem_sw[par]).wait()

    return sc_kernel(x, nodes, nbs_flat, pad_n, pad_nb)


def _tc_combine_body(self_ref, pooled_ref, ws_ref, wn_ref, b_ref, o_ref):
    acc = jnp.dot(self_ref[...], ws_ref[...], preferred_element_type=jnp.float32)
    acc = acc + jnp.dot(pooled_ref[...], wn_ref[...],
                        preferred_element_type=jnp.float32)
    o_ref[...] = jnp.maximum(acc + b_ref[...], 0.0)


def _tc_combine(self_rows, pooled, ws, wn_scaled, bias2d):
    bk = 2000
    grid = (B_REAL // bk,)
    return pl.pallas_call(
        _tc_combine_body,
        grid=grid,
        in_specs=[
            pl.BlockSpec((bk, D), lambda i: (i, 0)),
            pl.BlockSpec((bk, D), lambda i: (i, 0)),
            pl.BlockSpec((D, D), lambda i: (0, 0)),
            pl.BlockSpec((D, D), lambda i: (0, 0)),
            pl.BlockSpec((1, D), lambda i: (0, 0)),
        ],
        out_specs=pl.BlockSpec((bk, D), lambda i: (i, 0)),
        out_shape=jax.ShapeDtypeStruct((B_REAL, D), jnp.float32),
    )(self_rows, pooled, ws, wn_scaled, bias2d)


def kernel(x, nodes, nbs_idx, self_weight, nb_weight, bias):
    n = x.shape[0]
    pad_n = jnp.asarray(np.arange(BP - B_REAL, dtype=np.int32) % n)
    pad_nb = jnp.asarray(
        (np.arange((BP - B_REAL) * K_NBS, dtype=np.int32) % n)
        .reshape(BP - B_REAL, K_NBS))
    self_rows, pooled = _sc_gather_pool(
        x, nodes.astype(jnp.int32), nbs_idx.astype(jnp.int32),
        pad_n, pad_nb)
    return _tc_combine(self_rows, pooled, self_weight,
                       nb_weight * (1.0 / K_NBS), bias.reshape(1, D))


# speedup vs baseline: 5.4517x; 1.0157x over previous
import functools

import jax
import jax.numpy as jnp
from jax import lax
from jax.experimental import pallas as pl
from jax.experimental.pallas import tpu as pltpu
from jax.experimental.pallas import tpu_sc as plsc

D = 128
K_NBS = 32
NS = 16
NW = 2 * NS
CHUNK = 8
VPR = D // 16
ROWS = CHUNK * K_NBS
B_REAL = 10000
BP = 10240
Q = BP // NW
Q_TAIL = B_REAL - (NW - 1) * Q


def _accumulate_chunk(rows_v, pooled_v):
    for t in range(CHUNK):
        r0 = t * K_NBS

        def nb_body(i, a):
            for u in range(4):
                r = r0 + i * 4 + u
                a = tuple(a[v] + rows_v[r, pl.ds(v * 16, 16)]
                          for v in range(VPR))
            return a

        zero = jnp.zeros((16,), jnp.float32)
        accs = lax.fori_loop(0, K_NBS // 4, nb_body, (zero,) * VPR)
        for v in range(VPR):
            pooled_v[t, pl.ds(v * 16, 16)] = accs[v]


def _sc_gather_pool(x, nodes, nbs_flat):
    mesh = plsc.VectorSubcoreMesh(core_axis_name="c", subcore_axis_name="s")

    @functools.partial(
        pl.kernel,
        mesh=mesh,
        out_type=[
            jax.ShapeDtypeStruct((BP, D), jnp.float32),
            jax.ShapeDtypeStruct((BP, D), jnp.float32),
        ],
        scratch_types=[
            pltpu.VMEM((Q,), jnp.int32),
            pltpu.VMEM((Q, K_NBS), jnp.int32),
            pltpu.VMEM((Q * K_NBS,), jnp.int32),
            pltpu.VMEM((ROWS, D), jnp.float32),
            pltpu.VMEM((ROWS, D), jnp.float32),
            pltpu.VMEM((CHUNK, D), jnp.float32),
            pltpu.VMEM((CHUNK, D), jnp.float32),
            pltpu.VMEM((CHUNK, D), jnp.float32),
            pltpu.VMEM((CHUNK, D), jnp.float32),
            pltpu.SemaphoreType.DMA,
            pltpu.SemaphoreType.DMA,
            pltpu.SemaphoreType.DMA,
            pltpu.SemaphoreType.DMA,
            pltpu.SemaphoreType.DMA,
            pltpu.SemaphoreType.DMA,
            pltpu.SemaphoreType.DMA,
            pltpu.SemaphoreType.DMA,
        ],
    )
    def sc_kernel(x_hbm, nodes_hbm, nbs_hbm,
                  self_hbm, pooled_hbm,
                  nid_v, nbid2_v, nbid_v, rows0, rows1,
                  pooled0, pooled1, selfb0, selfb1,
                  sem_g0, sem_g1, sem_p0, sem_p1,
                  sem_sg0, sem_sg1, sem_sw0, sem_sw1):
        cid = lax.axis_index("c")
        sid = lax.axis_index("s")
        wid = sid * 2 + cid
        wbase = wid * Q
        rows = (rows0, rows1)
        pooled = (pooled0, pooled1)
        selfb = (selfb0, selfb1)
        sem_g = (sem_g0, sem_g1)
        sem_p = (sem_p0, sem_p1)
        sem_sg = (sem_sg0, sem_sg1)
        sem_sw = (sem_sw0, sem_sw1)

        is_tail = wid == NW - 1

        @pl.when(jnp.logical_not(is_tail))
        def _():
            pltpu.sync_copy(nodes_hbm.at[pl.ds(wbase, Q)], nid_v)
            pltpu.sync_copy(nbs_hbm.at[pl.ds(wbase, Q)], nbid2_v)

        @pl.when(is_tail)
        def _():
            tb = (NW - 1) * Q
            pltpu.sync_copy(nodes_hbm.at[pl.ds(tb, Q_TAIL)],
                            nid_v.at[pl.ds(0, Q_TAIL)])
            pltpu.sync_copy(nbs_hbm.at[pl.ds(tb, Q_TAIL)],
                            nbid2_v.at[pl.ds(0, Q_TAIL)])
            lanes = lax.iota(jnp.int32, 16)

            def pad_nid(j, carry):
                nid_v[pl.ds(Q_TAIL + j * 16, 16)] = lanes + j * 16
                return carry

            lax.fori_loop(0, (Q - Q_TAIL) // 16, pad_nid, 0)

            def pad_nbid(j, carry):
                for h in range(K_NBS // 16):
                    nbid2_v[Q_TAIL + j, pl.ds(h * 16, 16)] = (
                        lanes + (j * K_NBS + h * 16))
                return carry

            lax.fori_loop(0, Q - Q_TAIL, pad_nbid, 0)

        def flatten_ids(lo, hi):
            def body(q, carry):
                for h in range(K_NBS // 16):
                    nbid_v[pl.ds(q * K_NBS + h * 16, 16)] = (
                        nbid2_v[q, pl.ds(h * 16, 16)])
                return carry

            lax.fori_loop(lo, hi, body, 0)

        def start_gathers(c, par):
            pltpu.async_copy(
                x_hbm.at[nbid_v.at[pl.ds(c * ROWS, ROWS)]],
                rows[par], sem_g[par])
            pltpu.async_copy(
                x_hbm.at[nid_v.at[pl.ds(c * CHUNK, CHUNK)]],
                selfb[par], sem_sg[par])

        def finish_chunk(c, par, first):
            pltpu.make_async_copy(
                x_hbm.at[nbid_v.at[pl.ds(0, ROWS)]],
                rows[par], sem_g[par]).wait()

            @pl.when(jnp.logical_not(first))
            def _():
                pltpu.make_async_copy(
                    pooled[par], pooled_hbm.at[pl.ds(wbase, CHUNK)],
                    sem_p[par]).wait()

            _accumulate_chunk(rows[par], pooled[par])
            pltpu.async_copy(
                pooled[par],
                pooled_hbm.at[pl.ds(wbase + c * CHUNK, CHUNK)],
                sem_p[par])
            pltpu.make_async_copy(
                x_hbm.at[nid_v.at[pl.ds(0, CHUNK)]],
                selfb[par], sem_sg[par]).wait()

            @pl.when(jnp.logical_not(first))
            def _():
                pltpu.make_async_copy(
                    selfb[par], self_hbm.at[pl.ds(wbase, CHUNK)],
                    sem_sw[par]).wait()

            pltpu.async_copy(
                selfb[par],
                self_hbm.at[pl.ds(wbase + c * CHUNK, CHUNK)],
                sem_sw[par])

        n_chunks = Q // CHUNK
        flatten_ids(0, CHUNK)
        start_gathers(0, 0)
        flatten_ids(CHUNK, Q)

        def pair_body(p, carry):
            c0 = p * 2

            @pl.when(c0 + 1 < n_chunks)
            def _():
                start_gathers(c0 + 1, 1)

            finish_chunk(c0, 0, p == 0)

            @pl.when(c0 + 2 < n_chunks)
            def _():
                start_gathers(c0 + 2, 0)

            finish_chunk(c0 + 1, 1, p == 0)
            return carry

        lax.fori_loop(0, n_chunks // 2, pair_body, 0)
        for par in range(2):
            pltpu.make_async_copy(
                pooled[par], pooled_hbm.at[pl.ds(wbase, CHUNK)],
                sem_p[par]).wait()
            pltpu.make_async_copy(
                selfb[par], self_hbm.at[pl.ds(wbase, CHUNK)],
                sem_sw[par]).wait()

    return sc_kernel(x, nodes, nbs_flat)


def _tc_combine_body(self_ref, pooled_ref, ws_ref, wn_ref, b_ref, o_ref):
    acc = jnp.dot(self_ref[...], ws_ref[...], preferred_element_type=jnp.float32)
    acc = acc + jnp.dot(pooled_ref[...], wn_ref[...],
                        preferred_element_type=jnp.float32)
    o_ref[...] = jnp.maximum(acc + b_ref[...], 0.0)


def _tc_combine(self_rows, pooled, ws, wn_scaled, bias2d):
    bk = 2000
    grid = (B_REAL // bk,)
    return pl.pallas_call(
        _tc_combine_body,
        grid=grid,
        in_specs=[
            pl.BlockSpec((bk, D), lambda i: (i, 0)),
            pl.BlockSpec((bk, D), lambda i: (i, 0)),
            pl.BlockSpec((D, D), lambda i: (0, 0)),
            pl.BlockSpec((D, D), lambda i: (0, 0)),
            pl.BlockSpec((1, D), lambda i: (0, 0)),
        ],
        out_specs=pl.BlockSpec((bk, D), lambda i: (i, 0)),
        out_shape=jax.ShapeDtypeStruct((B_REAL, D), jnp.float32),
    )(self_rows, pooled, ws, wn_scaled, bias2d)


def kernel(x, nodes, nbs_idx, self_weight, nb_weight, bias):
    self_rows, pooled = _sc_gather_pool(
        x, nodes.astype(jnp.int32), nbs_idx.astype(jnp.int32))
    return _tc_combine(self_rows, pooled, self_weight,
                       nb_weight * (1.0 / K_NBS), bias.reshape(1, D))
